# baseline (speedup 1.0000x reference)
import jax
import jax.numpy as jnp
from jax.experimental import pallas as pl
from jax.experimental.pallas import tpu as pltpu

_BR = 256
_NS = 2


def _dot(a, b):
    return jax.lax.dot_general(
        a, b, (((1,), (0,)), ((), ())),
        preferred_element_type=jnp.float32,
        precision=jax.lax.Precision.DEFAULT)


def _gcn_kernel(x_ref, w_ref, *rest):
    adj_refs = rest[:_NS]
    bias_ref, out_ref, sup_ref = rest[_NS:]

    @pl.when(pl.program_id(0) == 0)
    def _():
        sup_ref[...] = _dot(x_ref[...], w_ref[...]).astype(jnp.bfloat16)

    for s in range(_NS):
        out_ref[s * _BR:(s + 1) * _BR, :] = (
            _dot(adj_refs[s][...], sup_ref[...]) + bias_ref[...])


def kernel(x, adj, weight, bias):
    n, d_in = x.shape
    d_out = weight.shape[1]
    bias2d = bias.reshape(1, d_out)

    def adj_spec(s):
        return pl.BlockSpec((_BR, n), lambda i, s=s: (_NS * i + s, 0))

    return pl.pallas_call(
        _gcn_kernel,
        grid=(pl.cdiv(n, _NS * _BR),),
        in_specs=[
            pl.BlockSpec((n, d_in), lambda i: (0, 0)),
            pl.BlockSpec((d_in, d_out), lambda i: (0, 0)),
        ] + [adj_spec(s) for s in range(_NS)] + [
            pl.BlockSpec((1, d_out), lambda i: (0, 0)),
        ],
        out_specs=pl.BlockSpec((_NS * _BR, d_out), lambda i: (i, 0)),
        out_shape=jax.ShapeDtypeStruct((n, d_out), jnp.float32),
        scratch_shapes=[pltpu.VMEM((n, d_out), jnp.bfloat16)],
    )(x, weight, *([adj] * _NS), bias2d)
